# baseline (speedup 1.0000x reference)
import jax
import jax.numpy as jnp
from jax.experimental import pallas as pl
from jax.experimental.pallas import tpu as pltpu

_TOPK = 8
_S, _B, _C, _D = 2048, 4, 64, 1024
_SC = 512
_NCHUNK = _S // _SC


def _eighth_largest(s):
    neg_inf = jnp.float32(-jnp.inf)
    shp = (s.shape[0], 1, s.shape[2])
    thr = jnp.full(shp, jnp.inf, jnp.float32)
    ans = jnp.full(shp, -jnp.inf, jnp.float32)
    k = jnp.full(shp, _TOPK, jnp.int32)
    done = jnp.zeros(shp, jnp.bool_)
    for _ in range(_TOPK):
        cand = jnp.where(s < thr, s, neg_inf)
        m = jnp.max(cand, axis=1, keepdims=True)
        c = jnp.sum((s == m).astype(jnp.int32), axis=1, keepdims=True)
        newly = jnp.logical_and(jnp.logical_not(done), k <= c)
        ans = jnp.where(newly, m, ans)
        cont = jnp.logical_not(jnp.logical_or(done, newly))
        k = jnp.where(cont, k - c, k)
        thr = jnp.where(cont, m, thr)
        done = jnp.logical_or(done, newly)
    return ans


def _fused(sim_ref, x_ref, pad_ref, pl_ref, out_ref, m_ref, den_ref):
    j = pl.program_id(0)
    neg_inf = jnp.float32(-jnp.inf)
    s = sim_ref[...]
    pad = pad_ref[...]
    s = jnp.where(pad[:, None, :] > 0, neg_inf, s)
    s = jnp.where(s > 0, s, neg_inf)
    t = _eighth_largest(s)
    masked = jnp.where(s >= t, s, neg_inf)
    pl_ref[...] = (masked > 0).astype(jnp.float32)
    m_chunk = jnp.maximum(jnp.max(masked, axis=2), 0.0)

    @pl.when(j == 0)
    def _init():
        m_ref[...] = jnp.zeros((_B, _C), jnp.float32)
        den_ref[...] = jnp.zeros((_B, _C), jnp.float32)
        out_ref[...] = jnp.zeros((_C, _B * _D), jnp.float32)

    m_old = m_ref[...]
    m_new = jnp.maximum(m_old, m_chunk)
    scale = jnp.exp(m_old - m_new)
    p = jnp.exp(masked - m_new[:, :, None])
    den_ref[...] = den_ref[...] * scale + jnp.sum(p, axis=2)
    m_ref[...] = m_new
    x2 = x_ref[...]
    for b in range(_B):
        part = jax.lax.dot(
            p[b], x2[:, b * _D:(b + 1) * _D],
            precision=jax.lax.Precision.HIGHEST,
            preferred_element_type=jnp.float32)
        out_ref[:, b * _D:(b + 1) * _D] = (
            out_ref[:, b * _D:(b + 1) * _D] * scale[b][:, None] + part)

    @pl.when(j == _NCHUNK - 1)
    def _fin():
        den = den_ref[...]
        inv = jnp.where(den > 0, 1.0 / den, 0.0)
        for b in range(_B):
            out_ref[:, b * _D:(b + 1) * _D] = (
                out_ref[:, b * _D:(b + 1) * _D] * inv[b][:, None])


def kernel(x, token_concept_embedding, key_padding_mask):
    simT = jnp.transpose(token_concept_embedding, (1, 2, 0))
    x2 = x.reshape(_S, _B * _D)
    padf = key_padding_mask.astype(jnp.float32)
    plT, out = pl.pallas_call(
        _fused,
        grid=(_NCHUNK,),
        in_specs=[
            pl.BlockSpec((_B, _C, _SC), lambda j: (0, 0, j)),
            pl.BlockSpec((_SC, _B * _D), lambda j: (j, 0)),
            pl.BlockSpec((_B, _SC), lambda j: (0, j)),
        ],
        out_specs=[
            pl.BlockSpec((_B, _C, _SC), lambda j: (0, 0, j)),
            pl.BlockSpec((_C, _B * _D), lambda j: (0, 0)),
        ],
        out_shape=[
            jax.ShapeDtypeStruct((_B, _C, _S), jnp.float32),
            jax.ShapeDtypeStruct((_C, _B * _D), jnp.float32),
        ],
        scratch_shapes=[
            pltpu.VMEM((_B, _C), jnp.float32),
            pltpu.VMEM((_B, _C), jnp.float32),
        ],
        compiler_params=pltpu.CompilerParams(
            dimension_semantics=("arbitrary",),
        ),
    )(simT, x2, padf)
    merge_val = out.reshape(_C, _B, _D)
    pseudo_label = jnp.transpose(plT, (2, 0, 1))
    return merge_val, pseudo_label
